# baseline (speedup 1.0000x reference)
import jax
import jax.numpy as jnp
from jax.experimental import pallas as pl
from jax.experimental.pallas import tpu as pltpu


def _make_body(tb, nt, n_vars):
    total = n_vars * nt

    def _copy(q_hbm, buf, sems, stepno, slot):
        vp = stepno % n_vars
        ip = stepno // n_vars
        return pltpu.make_async_copy(
            q_hbm.at[pl.ds(ip * tb, tb), vp], buf.at[slot], sems.at[slot])

    def _body(q_hbm, it_ref, out_ref, buf, sems):
        i = pl.program_id(0)
        v = pl.program_id(1)
        step = i * n_vars + v
        nbuf = buf.shape[0]
        slot = jax.lax.rem(step, nbuf)

        @pl.when(step == 0)
        def _prime():
            out_ref[...] = jnp.zeros_like(out_ref)
            for k in range(nbuf - 1):
                _copy(q_hbm, buf, sems, k, k).start()

        @pl.when(step + (nbuf - 1) < total)
        def _prefetch():
            _copy(q_hbm, buf, sems, step + (nbuf - 1),
                  jax.lax.rem(step + (nbuf - 1), nbuf)).start()

        _copy(q_hbm, buf, sems, step, slot).wait()

        n_mem = it_ref.shape[1]
        c = it_ref.shape[2]
        q = buf[slot]
        it = it_ref[v]
        qb = q.astype(jnp.bfloat16)
        st = jax.lax.dot_general(
            it.astype(jnp.bfloat16), qb,
            (((1,), (1,)), ((), ())),
            preferred_element_type=jnp.float32)
        row = jax.lax.broadcasted_iota(jnp.int32, (n_mem, 1), 0)
        code = (n_mem - 1) - row
        b = jax.lax.bitcast_convert_type(st, jnp.int32)
        s_emb = jax.lax.bitcast_convert_type((b & (-128)) | code,
                                             jnp.float32)
        m3 = s_emb.reshape(n_mem // 8, 8, tb)
        cmax = jnp.max(m3, axis=0)
        sh = 1
        while sh < 8:
            cmax = jnp.maximum(cmax, pltpu.roll(cmax, sh, 0))
            sh *= 2
        oh = m3 == cmax[None]
        it2 = (it * it).astype(jnp.bfloat16)
        nsqc = jax.lax.dot_general(
            it2, jnp.ones((8, c), jnp.bfloat16),
            (((1,), (1,)), ((), ())),
            preferred_element_type=jnp.float32)
        nsq3 = nsqc[:, :1].reshape(n_mem // 8, 8, 1)
        sel = jnp.where(oh, nsq3 - 2.0 * m3, 0.0)
        sel8 = jnp.sum(sel, axis=0)
        sel128 = jnp.sum(sel8.reshape(8, tb // 128, 128), axis=1)
        qq8 = jnp.sum((q * q).reshape(tb // 8, 8, c), axis=0)
        qq8p = jnp.concatenate(
            [qq8, jnp.zeros((8, 128 - c), jnp.float32)], axis=1)
        out_ref[...] += sel128 + qq8p

    return _body


def kernel(queries, items):
    t, n_vars, c = queries.shape
    n_mem = items.shape[1]
    tb = 512
    nt = t // tb
    part = pl.pallas_call(
        _make_body(tb, nt, n_vars),
        grid=(nt, n_vars),
        in_specs=[
            pl.BlockSpec(memory_space=pltpu.MemorySpace.HBM),
            pl.BlockSpec((n_vars, n_mem, c), lambda i, v: (0, 0, 0)),
        ],
        out_specs=pl.BlockSpec((8, 128), lambda i, v: (0, 0)),
        out_shape=jax.ShapeDtypeStruct((8, 128), jnp.float32),
        scratch_shapes=[
            pltpu.VMEM((12, tb, c), jnp.float32),
            pltpu.SemaphoreType.DMA((12,)),
        ],
    )(queries, items)
    return jnp.sum(part) / (t * n_vars * c)


# speedup vs baseline: 1.5410x; 1.5410x over previous
import jax
import jax.numpy as jnp
from jax.experimental import pallas as pl
from jax.experimental.pallas import tpu as pltpu


def _make_body(tb, nt, n_vars):

    def _copy(q_hbm, buf, sems, ip, par, v):
        return pltpu.make_async_copy(
            q_hbm.at[pl.ds(ip * tb, tb), v], buf.at[par, v],
            sems.at[par, v])

    def _body(q_hbm, it_ref, out_ref, buf, sems):
        i = pl.program_id(0)
        par = jax.lax.rem(i, 2)
        nxt = jax.lax.rem(i + 1, 2)

        @pl.when(i == 0)
        def _prime():
            out_ref[...] = jnp.zeros_like(out_ref)
            for v in range(n_vars):
                _copy(q_hbm, buf, sems, i, par, v).start()

        @pl.when(i + 1 < nt)
        def _prefetch():
            for v in range(n_vars):
                _copy(q_hbm, buf, sems, i + 1, nxt, v).start()

        n_mem = it_ref.shape[1]
        c = it_ref.shape[2]
        row = jax.lax.broadcasted_iota(jnp.int32, (n_mem, 1), 0)
        code = (n_mem - 1) - row
        acc = jnp.zeros((8, 128), jnp.float32)
        for v in range(n_vars):
            _copy(q_hbm, buf, sems, i, par, v).wait()
        for v in range(n_vars):
            q = buf[par, v]
            it = it_ref[v]
            st = jax.lax.dot_general(
                it.astype(jnp.bfloat16), q.astype(jnp.bfloat16),
                (((1,), (1,)), ((), ())),
                preferred_element_type=jnp.float32)
            b = jax.lax.bitcast_convert_type(st, jnp.int32)
            s_emb = jax.lax.bitcast_convert_type((b & (-128)) | code,
                                                 jnp.float32)
            m3 = s_emb.reshape(n_mem // 8, 8, tb)
            cmax = jnp.max(m3, axis=0)
            sh = 1
            while sh < 8:
                cmax = jnp.maximum(cmax, pltpu.roll(cmax, sh, 0))
                sh *= 2
            oh = m3 == cmax[None]
            it2 = (it * it).astype(jnp.bfloat16)
            nsqc = jax.lax.dot_general(
                it2, jnp.ones((8, c), jnp.bfloat16),
                (((1,), (1,)), ((), ())),
                preferred_element_type=jnp.float32)
            nsq3 = nsqc[:, :1].reshape(n_mem // 8, 8, 1)
            sel = jnp.where(oh, nsq3 - 2.0 * m3, 0.0)
            sel8 = jnp.sum(sel, axis=0)
            sel128 = jnp.sum(sel8.reshape(8, tb // 128, 128), axis=1)
            qq8 = jnp.sum((q * q).reshape(tb // 8, 8, c), axis=0)
            qq8p = jnp.concatenate(
                [qq8, jnp.zeros((8, 128 - c), jnp.float32)], axis=1)
            acc = acc + sel128 + qq8p
        out_ref[...] += acc

    return _body


def kernel(queries, items):
    t, n_vars, c = queries.shape
    n_mem = items.shape[1]
    tb = 512
    nt = t // tb
    part = pl.pallas_call(
        _make_body(tb, nt, n_vars),
        grid=(nt,),
        in_specs=[
            pl.BlockSpec(memory_space=pltpu.MemorySpace.HBM),
            pl.BlockSpec((n_vars, n_mem, c), lambda i: (0, 0, 0)),
        ],
        out_specs=pl.BlockSpec((8, 128), lambda i: (0, 0)),
        out_shape=jax.ShapeDtypeStruct((8, 128), jnp.float32),
        scratch_shapes=[
            pltpu.VMEM((2, n_vars, tb, c), jnp.float32),
            pltpu.SemaphoreType.DMA((2, n_vars)),
        ],
    )(queries, items)
    return jnp.sum(part) / (t * n_vars * c)
